# Initial kernel scaffold; baseline (speedup 1.0000x reference)
import jax
import jax.numpy as jnp
from jax.experimental import pallas as pl


def kernel(x, edge_index, W1, b1, W2, b2, Wfc, bfc):
    raise NotImplementedError("write your pallas kernel here")


# speedup vs baseline: 21.0465x; 21.0465x over previous
import functools

import jax
import jax.numpy as jnp
from jax import lax
from jax.experimental import pallas as pl
from jax.experimental.pallas import tpu as pltpu
from jax.experimental.pallas import tpu_sc as plsc

N = 10000
D = 128
NPAD = 10240
CHUNKS = 79
EPAD = 32 * CHUNKS * 128
SHARE = NPAD // 16
BLK = 1024
GRID = NPAD // BLK

_mesh = plsc.VectorSubcoreMesh(core_axis_name="c", subcore_axis_name="s")


DEGW = 16


@functools.partial(
    pl.kernel,
    out_type=jax.ShapeDtypeStruct((2, NPAD, DEGW), jnp.float32),
    mesh=_mesh,
    scratch_types=[
        pltpu.VMEM((CHUNKS, 128), jnp.int32),
        pltpu.VMEM((128, DEGW), jnp.float32),
        pltpu.VMEM_SHARED((NPAD, DEGW), jnp.float32),
    ],
)
def _sc_degree(dstp, zeros_col, ones_col, out, idx_v, ones_v, deg_sh):
    c = lax.axis_index("c")
    s = lax.axis_index("s")
    wid = c * 16 + s
    pltpu.sync_copy(dstp.at[wid], idx_v)
    pltpu.sync_copy(ones_col, ones_v)
    pltpu.sync_copy(
        zeros_col.at[pl.ds(s * SHARE, SHARE)], deg_sh.at[pl.ds(s * SHARE, SHARE)]
    )
    plsc.subcore_barrier()
    for j in range(CHUNKS):
        pltpu.sync_copy(ones_v, deg_sh.at[idx_v.at[j]], add=True)
    plsc.subcore_barrier()
    pltpu.sync_copy(
        deg_sh.at[pl.ds(s * SHARE, SHARE)], out.at[c].at[pl.ds(s * SHARE, SHARE)]
    )


@functools.partial(
    pl.kernel,
    out_type=jax.ShapeDtypeStruct((2, NPAD, D), jnp.float32),
    mesh=_mesh,
    scratch_types=[
        pltpu.VMEM((CHUNKS, 128), jnp.int32),
        pltpu.VMEM((CHUNKS, 128), jnp.int32),
        pltpu.VMEM((128, D), jnp.float32),
        pltpu.VMEM_SHARED((NPAD, D), jnp.float32),
        pltpu.SemaphoreType.DMA,
    ],
)
def _sc_propagate(g, srcp, dstp, zeros2d, out, isrc, idst, rows, s_sh, sem):
    c = lax.axis_index("c")
    s = lax.axis_index("s")
    wid = c * 16 + s
    pltpu.sync_copy(srcp.at[wid], isrc)
    pltpu.sync_copy(dstp.at[wid], idst)
    pltpu.sync_copy(
        zeros2d.at[pl.ds(s * SHARE, SHARE)], s_sh.at[pl.ds(s * SHARE, SHARE)]
    )
    plsc.subcore_barrier()
    for j in range(CHUNKS):
        pltpu.async_copy(g.at[isrc.at[j]], rows, sem).wait()
        pltpu.sync_copy(rows, s_sh.at[idst.at[j]], add=True)
    plsc.subcore_barrier()
    pltpu.sync_copy(
        s_sh.at[pl.ds(s * SHARE, SHARE)], out.at[c].at[pl.ds(s * SHARE, SHARE)]
    )


def _tc1_body(x_ref, w1_ref, d0_ref, d1_ref, g_ref):
    dis = lax.rsqrt(d0_ref[:, :1] + d1_ref[:, :1] + 1.0)
    h = jnp.dot(x_ref[...], w1_ref[...], preferred_element_type=jnp.float32)
    g_ref[...] = h * dis


_tc1 = pl.pallas_call(
    _tc1_body,
    grid=(GRID,),
    in_specs=[
        pl.BlockSpec((BLK, D), lambda i: (i, 0)),
        pl.BlockSpec((D, D), lambda i: (0, 0)),
        pl.BlockSpec((BLK, DEGW), lambda i: (i, 0)),
        pl.BlockSpec((BLK, DEGW), lambda i: (i, 0)),
    ],
    out_specs=pl.BlockSpec((BLK, D), lambda i: (i, 0)),
    out_shape=jax.ShapeDtypeStruct((NPAD, D), jnp.float32),
)


def _tc2_body(d0_ref, d1_ref, s_ref, g1_ref, b1_ref, w2_ref, g2_ref):
    dis = lax.rsqrt(d0_ref[:, :1] + d1_ref[:, :1] + 1.0)
    stot = s_ref[0] + s_ref[1] + g1_ref[...]
    h1 = jnp.maximum(stot * dis + b1_ref[...], 0.0)
    g2_ref[...] = jnp.dot(h1, w2_ref[...], preferred_element_type=jnp.float32) * dis


_tc2 = pl.pallas_call(
    _tc2_body,
    grid=(GRID,),
    in_specs=[
        pl.BlockSpec((BLK, DEGW), lambda i: (i, 0)),
        pl.BlockSpec((BLK, DEGW), lambda i: (i, 0)),
        pl.BlockSpec((2, BLK, D), lambda i: (0, i, 0)),
        pl.BlockSpec((BLK, D), lambda i: (i, 0)),
        pl.BlockSpec((1, D), lambda i: (0, 0)),
        pl.BlockSpec((D, D), lambda i: (0, 0)),
    ],
    out_specs=pl.BlockSpec((BLK, D), lambda i: (i, 0)),
    out_shape=jax.ShapeDtypeStruct((NPAD, D), jnp.float32),
)


def _tc3_body(d0_ref, d1_ref, s_ref, g2_ref, b2_ref, wfc_ref, bfc_ref, out_ref, acc_ref):
    i = pl.program_id(0)
    dis = lax.rsqrt(d0_ref[:, :1] + d1_ref[:, :1] + 1.0)
    stot = s_ref[0] + s_ref[1] + g2_ref[...]
    h2 = jnp.maximum(stot * dis + b2_ref[...], 0.0)
    row = i * BLK + lax.broadcasted_iota(jnp.int32, (BLK, 1), 0)
    h2 = jnp.where(row < N, h2, 0.0)

    @pl.when(i == 0)
    def _init():
        acc_ref[...] = jnp.zeros_like(acc_ref)

    acc_ref[...] += jnp.sum(h2, axis=0, keepdims=True)

    @pl.when(i == pl.num_programs(0) - 1)
    def _final():
        pooled = acc_ref[...] * (1.0 / N)
        out_ref[...] = (
            jnp.dot(pooled, wfc_ref[...], preferred_element_type=jnp.float32)
            + bfc_ref[...]
        )


_tc3 = pl.pallas_call(
    _tc3_body,
    grid=(GRID,),
    in_specs=[
        pl.BlockSpec((BLK, DEGW), lambda i: (i, 0)),
        pl.BlockSpec((BLK, DEGW), lambda i: (i, 0)),
        pl.BlockSpec((2, BLK, D), lambda i: (0, i, 0)),
        pl.BlockSpec((BLK, D), lambda i: (i, 0)),
        pl.BlockSpec((1, D), lambda i: (0, 0)),
        pl.BlockSpec((D, D), lambda i: (0, 0)),
        pl.BlockSpec((1, D), lambda i: (0, 0)),
    ],
    out_specs=pl.BlockSpec((1, D), lambda i: (0, 0)),
    out_shape=jax.ShapeDtypeStruct((1, D), jnp.float32),
    scratch_shapes=[pltpu.VMEM((1, D), jnp.float32)],
)


def kernel(x, edge_index, W1, b1, W2, b2, Wfc, bfc):
    src = edge_index[0].astype(jnp.int32)
    dst = edge_index[1].astype(jnp.int32)
    npad_e = EPAD - src.shape[0]
    pad_ar = jnp.arange(npad_e, dtype=jnp.int32)
    pad_src = pad_ar % N
    pad_dst = N + pad_ar % (NPAD - N)
    srcp = jnp.concatenate([src, pad_src]).reshape(32, CHUNKS, 128)
    dstp = jnp.concatenate([dst, pad_dst]).reshape(32, CHUNKS, 128)

    zeros_col = jnp.zeros((NPAD, DEGW), jnp.float32)
    ones_col = jnp.ones((128, DEGW), jnp.float32)
    zeros2d = jnp.zeros((NPAD, D), jnp.float32)
    x_pad = jnp.concatenate([x, jnp.zeros((NPAD - N, D), jnp.float32)])

    degp = _sc_degree(dstp, zeros_col, ones_col)
    d0 = degp[0]
    d1 = degp[1]

    g1 = _tc1(x_pad, W1, d0, d1)
    s1 = _sc_propagate(g1, srcp, dstp, zeros2d)
    g2 = _tc2(d0, d1, s1, g1, b1.reshape(1, D), W2)
    s2 = _sc_propagate(g2, srcp, dstp, zeros2d)
    out = _tc3(d0, d1, s2, g2, b2.reshape(1, D), Wfc, bfc.reshape(1, D))
    return out.reshape(D)


# speedup vs baseline: 29.3992x; 1.3969x over previous
import functools

import jax
import jax.numpy as jnp
from jax import lax
from jax.experimental import pallas as pl
from jax.experimental.pallas import tpu as pltpu
from jax.experimental.pallas import tpu_sc as plsc

N = 10000
D = 128
NPAD = 10240
CHUNKS = 80
PCH = 40
EPAD = 32 * CHUNKS * 128
SHARE = NPAD // 16
BLK = 1024
GRID = NPAD // BLK

_mesh = plsc.VectorSubcoreMesh(core_axis_name="c", subcore_axis_name="s")


DEGW = 16


@functools.partial(
    pl.kernel,
    out_type=jax.ShapeDtypeStruct((2, NPAD, DEGW), jnp.float32),
    mesh=_mesh,
    scratch_types=[
        pltpu.VMEM((CHUNKS, 128), jnp.int32),
        pltpu.VMEM((128, DEGW), jnp.float32),
        pltpu.VMEM_SHARED((NPAD, DEGW), jnp.float32),
    ],
)
def _sc_degree(dstp, zeros_col, ones_col, out, idx_v, ones_v, deg_sh):
    c = lax.axis_index("c")
    s = lax.axis_index("s")
    wid = c * 16 + s
    pltpu.sync_copy(dstp.at[wid], idx_v)
    pltpu.sync_copy(ones_col, ones_v)
    pltpu.sync_copy(
        zeros_col.at[pl.ds(s * SHARE, SHARE)], deg_sh.at[pl.ds(s * SHARE, SHARE)]
    )
    plsc.subcore_barrier()
    for j in range(CHUNKS):
        pltpu.sync_copy(ones_v, deg_sh.at[idx_v.at[j]], add=True)
    plsc.subcore_barrier()
    pltpu.sync_copy(
        deg_sh.at[pl.ds(s * SHARE, SHARE)], out.at[c].at[pl.ds(s * SHARE, SHARE)]
    )


NBUF = 2


@functools.partial(
    pl.kernel,
    out_type=jax.ShapeDtypeStruct((2, NPAD, D), jnp.float32),
    mesh=_mesh,
    scratch_types=[
        pltpu.VMEM((PCH, 128), jnp.int32),
        pltpu.VMEM((PCH, 128), jnp.int32),
        pltpu.VMEM((NBUF, 128, D), jnp.float32),
        pltpu.VMEM_SHARED((NPAD, D), jnp.float32),
        [pltpu.SemaphoreType.DMA] * NBUF,
        [pltpu.SemaphoreType.DMA] * NBUF,
    ],
)
def _sc_propagate(g, srcp, dstp, zeros2d, out, isrc, idst, rows, s_sh, gsems, ssems):
    c = lax.axis_index("c")
    s = lax.axis_index("s")
    wid = c * 16 + s
    pltpu.sync_copy(
        zeros2d.at[pl.ds(s * SHARE, SHARE)], s_sh.at[pl.ds(s * SHARE, SHARE)]
    )
    plsc.subcore_barrier()
    for p in range(CHUNKS // PCH):
        pltpu.sync_copy(srcp.at[wid].at[pl.ds(p * PCH, PCH)], isrc)
        pltpu.sync_copy(dstp.at[wid].at[pl.ds(p * PCH, PCH)], idst)
        gd = [None] * NBUF
        sd = [None] * NBUF
        gd[0] = pltpu.async_copy(g.at[isrc.at[0]], rows.at[0], gsems[0])
        for j in range(PCH):
            b = j % NBUF
            nb = (j + 1) % NBUF
            if j + 1 < PCH:
                if sd[nb] is not None:
                    sd[nb].wait()
                gd[nb] = pltpu.async_copy(
                    g.at[isrc.at[j + 1]], rows.at[nb], gsems[nb]
                )
            gd[b].wait()
            sd[b] = pltpu.async_copy(
                rows.at[b], s_sh.at[idst.at[j]], ssems[b], add=True
            )
        for b in range(NBUF):
            if sd[b] is not None:
                sd[b].wait()
    plsc.subcore_barrier()
    pltpu.sync_copy(
        s_sh.at[pl.ds(s * SHARE, SHARE)], out.at[c].at[pl.ds(s * SHARE, SHARE)]
    )


def _tc1_body(x_ref, w1_ref, d0_ref, d1_ref, g_ref):
    dis = lax.rsqrt(d0_ref[:, :1] + d1_ref[:, :1] + 1.0)
    h = jnp.dot(x_ref[...], w1_ref[...], preferred_element_type=jnp.float32)
    g_ref[...] = h * dis


_tc1 = pl.pallas_call(
    _tc1_body,
    grid=(GRID,),
    in_specs=[
        pl.BlockSpec((BLK, D), lambda i: (i, 0)),
        pl.BlockSpec((D, D), lambda i: (0, 0)),
        pl.BlockSpec((BLK, DEGW), lambda i: (i, 0)),
        pl.BlockSpec((BLK, DEGW), lambda i: (i, 0)),
    ],
    out_specs=pl.BlockSpec((BLK, D), lambda i: (i, 0)),
    out_shape=jax.ShapeDtypeStruct((NPAD, D), jnp.float32),
)


def _tc2_body(d0_ref, d1_ref, s_ref, g1_ref, b1_ref, w2_ref, g2_ref):
    dis = lax.rsqrt(d0_ref[:, :1] + d1_ref[:, :1] + 1.0)
    stot = s_ref[0] + s_ref[1] + g1_ref[...]
    h1 = jnp.maximum(stot * dis + b1_ref[...], 0.0)
    g2_ref[...] = jnp.dot(h1, w2_ref[...], preferred_element_type=jnp.float32) * dis


_tc2 = pl.pallas_call(
    _tc2_body,
    grid=(GRID,),
    in_specs=[
        pl.BlockSpec((BLK, DEGW), lambda i: (i, 0)),
        pl.BlockSpec((BLK, DEGW), lambda i: (i, 0)),
        pl.BlockSpec((2, BLK, D), lambda i: (0, i, 0)),
        pl.BlockSpec((BLK, D), lambda i: (i, 0)),
        pl.BlockSpec((1, D), lambda i: (0, 0)),
        pl.BlockSpec((D, D), lambda i: (0, 0)),
    ],
    out_specs=pl.BlockSpec((BLK, D), lambda i: (i, 0)),
    out_shape=jax.ShapeDtypeStruct((NPAD, D), jnp.float32),
)


def _tc3_body(d0_ref, d1_ref, s_ref, g2_ref, b2_ref, wfc_ref, bfc_ref, out_ref, acc_ref):
    i = pl.program_id(0)
    dis = lax.rsqrt(d0_ref[:, :1] + d1_ref[:, :1] + 1.0)
    stot = s_ref[0] + s_ref[1] + g2_ref[...]
    h2 = jnp.maximum(stot * dis + b2_ref[...], 0.0)
    row = i * BLK + lax.broadcasted_iota(jnp.int32, (BLK, 1), 0)
    h2 = jnp.where(row < N, h2, 0.0)

    @pl.when(i == 0)
    def _init():
        acc_ref[...] = jnp.zeros_like(acc_ref)

    acc_ref[...] += jnp.sum(h2, axis=0, keepdims=True)

    @pl.when(i == pl.num_programs(0) - 1)
    def _final():
        pooled = acc_ref[...] * (1.0 / N)
        out_ref[...] = (
            jnp.dot(pooled, wfc_ref[...], preferred_element_type=jnp.float32)
            + bfc_ref[...]
        )


_tc3 = pl.pallas_call(
    _tc3_body,
    grid=(GRID,),
    in_specs=[
        pl.BlockSpec((BLK, DEGW), lambda i: (i, 0)),
        pl.BlockSpec((BLK, DEGW), lambda i: (i, 0)),
        pl.BlockSpec((2, BLK, D), lambda i: (0, i, 0)),
        pl.BlockSpec((BLK, D), lambda i: (i, 0)),
        pl.BlockSpec((1, D), lambda i: (0, 0)),
        pl.BlockSpec((D, D), lambda i: (0, 0)),
        pl.BlockSpec((1, D), lambda i: (0, 0)),
    ],
    out_specs=pl.BlockSpec((1, D), lambda i: (0, 0)),
    out_shape=jax.ShapeDtypeStruct((1, D), jnp.float32),
    scratch_shapes=[pltpu.VMEM((1, D), jnp.float32)],
)


def kernel(x, edge_index, W1, b1, W2, b2, Wfc, bfc):
    src = edge_index[0].astype(jnp.int32)
    dst = edge_index[1].astype(jnp.int32)
    npad_e = EPAD - src.shape[0]
    pad_ar = jnp.arange(npad_e, dtype=jnp.int32)
    pad_src = pad_ar % N
    pad_dst = N + pad_ar % (NPAD - N)
    srcp = jnp.concatenate([src, pad_src]).reshape(32, CHUNKS, 128)
    dstp = jnp.concatenate([dst, pad_dst]).reshape(32, CHUNKS, 128)

    zeros_col = jnp.zeros((NPAD, DEGW), jnp.float32)
    ones_col = jnp.ones((128, DEGW), jnp.float32)
    zeros2d = jnp.zeros((NPAD, D), jnp.float32)
    x_pad = jnp.concatenate([x, jnp.zeros((NPAD - N, D), jnp.float32)])

    degp = _sc_degree(dstp, zeros_col, ones_col)
    d0 = degp[0]
    d1 = degp[1]

    g1 = _tc1(x_pad, W1, d0, d1)
    s1 = _sc_propagate(g1, srcp, dstp, zeros2d)
    g2 = _tc2(d0, d1, s1, g1, b1.reshape(1, D), W2)
    s2 = _sc_propagate(g2, srcp, dstp, zeros2d)
    out = _tc3(d0, d1, s2, g2, b2.reshape(1, D), Wfc, bfc.reshape(1, D))
    return out.reshape(D)
